# baseline (speedup 1.0000x reference)
import functools

import jax
import jax.numpy as jnp
from jax import lax
from jax.experimental import pallas as pl
from jax.experimental.pallas import tpu as pltpu
from jax.experimental.pallas import tpu_sc as plsc

_NUM_SEGMENTS = 1024
_N = 320000
_D = 128

_NC = 2
_NS = 16
_L = 16
_NW = _NC * _NS
_CH = _N // _NW
_CHV = _CH // _L
_HIST = _L * _NUM_SEGMENTS

_GW = 128
_GROWS = 80
_GFIRE = 8


def _sc_partials(out_flat, batch_i32):
    mesh = plsc.VectorSubcoreMesh(core_axis_name="c", subcore_axis_name="s")

    @functools.partial(
        pl.kernel,
        out_type=(
            jax.ShapeDtypeStruct((_NW * _HIST // 128, 128), jnp.float32),
            jax.ShapeDtypeStruct((_NW * _HIST // 128, 128), jnp.float32),
        ),
        mesh=mesh,
        compiler_params=pltpu.CompilerParams(
            use_tc_tiling_on_sc=False, needs_layout_passes=False
        ),
        scratch_types=[
            pltpu.VMEM((_CH,), jnp.int32),
            pltpu.VMEM((_GROWS, _GW), jnp.int32),
            pltpu.VMEM((_GROWS, _GW), jnp.float32),
            pltpu.VMEM((_HIST // 128, 128), jnp.float32),
            pltpu.VMEM((_HIST // 128, 128), jnp.float32),
            pltpu.SemaphoreType.DMA,
            pltpu.SemaphoreType.DMA,
            pltpu.SemaphoreType.DMA,
        ],
    )
    def k(flat_hbm, ids_hbm, parts_s, parts_c,
          ids_v, idx_v, vals_g, acc_s, acc_c, sem_i, sem_a, sem_b):
        wid = lax.axis_index("s") * _NC + lax.axis_index("c")
        base = wid * _CH

        ids_cp = pltpu.async_copy(ids_hbm.at[pl.ds(base, _CH)], ids_v, sem_i)

        lane_iota = lax.iota(jnp.int32, _L)
        last = base + (_CH - 1)
        sems = (sem_a, sem_b)
        ngroups = _GROWS // _GFIRE

        def fill_group(g):
            for r in range(_GFIRE):
                j = g * _GFIRE + r
                for kk in range(_GW // _L):
                    elem = base + j * _GW + kk * _L + lane_iota
                    elem = jnp.minimum(elem, last)
                    idx_v[j, pl.ds(kk * _L, _L)] = elem * _D

        def fire_group(g):
            s = sems[g % 2]
            return [
                pltpu.async_copy(
                    flat_hbm.at[idx_v.at[g * _GFIRE + r]],
                    vals_g.at[g * _GFIRE + r], s)
                for r in range(_GFIRE)
            ]

        fill_group(0)
        inflight = {0: fire_group(0)}

        zeros = jnp.zeros((_L,), jnp.float32)

        def zero_body(i, _):
            acc_s[i // 8, pl.ds((i % 8) * _L, _L)] = zeros
            acc_c[i // 8, pl.ds((i % 8) * _L, _L)] = zeros
            return None

        lax.fori_loop(0, _HIST // _L, zero_body, None, unroll=4)

        ids_cp.wait()

        lane_off = lane_iota * _NUM_SEGMENTS
        ones = jnp.ones((_L,), jnp.float32)

        def acc_body(t, _):
            ids = ids_v[pl.ds(t * _L, _L)]
            vals = vals_g[t // 8, pl.ds((t % 8) * _L, _L)]
            addr = ids + lane_off
            row = lax.shift_right_logical(addr, 7)
            col = lax.bitwise_and(addr, 127)
            plsc.addupdate_scatter(acc_s, [row, col], vals)
            plsc.addupdate_scatter(acc_c, [row, col], ones)
            return None

        vpg = _GFIRE * _GW // _L
        for g in range(ngroups):
            if g + 1 < ngroups:
                fill_group(g + 1)
                inflight[g + 1] = fire_group(g + 1)
            for cp in inflight.pop(g):
                cp.wait()
            lax.fori_loop(g * vpg, min((g + 1) * vpg, _CHV),
                          acc_body, None, unroll=4)

        prow = wid * (_HIST // 128)
        pltpu.sync_copy(acc_s, parts_s.at[pl.ds(prow, _HIST // 128), :])
        pltpu.sync_copy(acc_c, parts_c.at[pl.ds(prow, _HIST // 128), :])

    return k(out_flat, batch_i32)


def _finish_body(ps_ref, pc_ref, o_ref):
    s = jnp.sum(ps_ref[...].reshape(_NW * _L, 8, 128), axis=0)
    c = jnp.sum(pc_ref[...].reshape(_NW * _L, 8, 128), axis=0)
    o_ref[...] = s / jnp.maximum(c, 1.0)


def kernel(outputs, batch, is_global):
    del is_global
    batch_i32 = batch.astype(jnp.int32)
    parts_s, parts_c = _sc_partials(outputs.reshape(_N * _D), batch_i32)
    score2d = pl.pallas_call(
        _finish_body,
        out_shape=jax.ShapeDtypeStruct((8, 128), jnp.float32),
    )(parts_s, parts_c)
    return score2d.reshape(_NUM_SEGMENTS)


# speedup vs baseline: 1.0236x; 1.0236x over previous
import functools

import jax
import jax.numpy as jnp
from jax import lax
from jax.experimental import pallas as pl
from jax.experimental.pallas import tpu as pltpu
from jax.experimental.pallas import tpu_sc as plsc

_NUM_SEGMENTS = 1024
_N = 320000
_D = 128

_NC = 2
_NS = 16
_L = 16
_NW = _NC * _NS
_CH = _N // _NW
_CHV = _CH // _L
_HIST = _L * _NUM_SEGMENTS

_GW = 128
_GROWS = 80
_GFIRE = 8


def _sc_partials(out_flat, batch_i32):
    mesh = plsc.VectorSubcoreMesh(core_axis_name="c", subcore_axis_name="s")

    @functools.partial(
        pl.kernel,
        out_type=(
            jax.ShapeDtypeStruct((_NW * _NUM_SEGMENTS // 128, 128), jnp.float32),
            jax.ShapeDtypeStruct((_NW * _NUM_SEGMENTS // 128, 128), jnp.float32),
        ),
        mesh=mesh,
        compiler_params=pltpu.CompilerParams(
            use_tc_tiling_on_sc=False, needs_layout_passes=False
        ),
        scratch_types=[
            pltpu.VMEM((_CH,), jnp.int32),
            pltpu.VMEM((_GROWS, _GW), jnp.int32),
            pltpu.VMEM((_GROWS, _GW), jnp.float32),
            pltpu.VMEM((_HIST // 128, 128), jnp.float32),
            pltpu.VMEM((_HIST // 128, 128), jnp.float32),
            pltpu.VMEM((_NUM_SEGMENTS // 128, 128), jnp.float32),
            pltpu.VMEM((_NUM_SEGMENTS // 128, 128), jnp.float32),
            pltpu.SemaphoreType.DMA,
            pltpu.SemaphoreType.DMA,
            pltpu.SemaphoreType.DMA,
        ],
    )
    def k(flat_hbm, ids_hbm, parts_s, parts_c,
          ids_v, idx_v, vals_g, acc_s, acc_c, red_s, red_c,
          sem_i, sem_a, sem_b):
        wid = lax.axis_index("s") * _NC + lax.axis_index("c")
        base = wid * _CH

        ids_cp = pltpu.async_copy(ids_hbm.at[pl.ds(base, _CH)], ids_v, sem_i)

        lane_iota = lax.iota(jnp.int32, _L)
        last = base + (_CH - 1)
        sems = (sem_a, sem_b)
        ngroups = _GROWS // _GFIRE

        def fill_group(g):
            for r in range(_GFIRE):
                j = g * _GFIRE + r
                for kk in range(_GW // _L):
                    elem = base + j * _GW + kk * _L + lane_iota
                    elem = jnp.minimum(elem, last)
                    idx_v[j, pl.ds(kk * _L, _L)] = elem * _D

        def fire_group(g):
            s = sems[g % 2]
            return [
                pltpu.async_copy(
                    flat_hbm.at[idx_v.at[g * _GFIRE + r]],
                    vals_g.at[g * _GFIRE + r], s)
                for r in range(_GFIRE)
            ]

        fill_group(0)
        inflight = {0: fire_group(0)}

        zeros = jnp.zeros((_L,), jnp.float32)

        def zero_body(i, _):
            acc_s[i // 8, pl.ds((i % 8) * _L, _L)] = zeros
            acc_c[i // 8, pl.ds((i % 8) * _L, _L)] = zeros
            return None

        lax.fori_loop(0, _HIST // _L, zero_body, None, unroll=4)

        ids_cp.wait()

        lane_off = lane_iota * _NUM_SEGMENTS
        ones = jnp.ones((_L,), jnp.float32)

        def acc_body(t, _):
            ids = ids_v[pl.ds(t * _L, _L)]
            vals = vals_g[t // 8, pl.ds((t % 8) * _L, _L)]
            addr = ids + lane_off
            row = lax.shift_right_logical(addr, 7)
            col = lax.bitwise_and(addr, 127)
            plsc.addupdate_scatter(acc_s, [row, col], vals)
            plsc.addupdate_scatter(acc_c, [row, col], ones)
            return None

        vpg = _GFIRE * _GW // _L
        for g in range(ngroups):
            if g + 1 < ngroups:
                fill_group(g + 1)
                inflight[g + 1] = fire_group(g + 1)
            for cp in inflight.pop(g):
                cp.wait()
            lax.fori_loop(g * vpg, min((g + 1) * vpg, _CHV),
                          acc_body, None, unroll=4)

        nblk = _NUM_SEGMENTS // 128

        def red_body(i, _):
            kk = i // 8
            jj = i % 8
            s = acc_s[kk, pl.ds(jj * _L, _L)]
            c = acc_c[kk, pl.ds(jj * _L, _L)]
            for l in range(1, _L):
                s = s + acc_s[l * nblk + kk, pl.ds(jj * _L, _L)]
                c = c + acc_c[l * nblk + kk, pl.ds(jj * _L, _L)]
            red_s[kk, pl.ds(jj * _L, _L)] = s
            red_c[kk, pl.ds(jj * _L, _L)] = c
            return None

        lax.fori_loop(0, nblk * 8, red_body, None)

        prow = wid * nblk
        pltpu.sync_copy(red_s, parts_s.at[pl.ds(prow, nblk), :])
        pltpu.sync_copy(red_c, parts_c.at[pl.ds(prow, nblk), :])

    return k(out_flat, batch_i32)


def _finish_body(ps_ref, pc_ref, o_ref):
    s = jnp.sum(ps_ref[...].reshape(_NW, 8, 128), axis=0)
    c = jnp.sum(pc_ref[...].reshape(_NW, 8, 128), axis=0)
    o_ref[...] = s / jnp.maximum(c, 1.0)


def kernel(outputs, batch, is_global):
    del is_global
    batch_i32 = batch.astype(jnp.int32)
    parts_s, parts_c = _sc_partials(outputs.reshape(_N * _D), batch_i32)
    score2d = pl.pallas_call(
        _finish_body,
        out_shape=jax.ShapeDtypeStruct((8, 128), jnp.float32),
    )(parts_s, parts_c)
    return score2d.reshape(_NUM_SEGMENTS)


# speedup vs baseline: 1.0531x; 1.0288x over previous
import functools

import jax
import jax.numpy as jnp
from jax import lax
from jax.experimental import pallas as pl
from jax.experimental.pallas import tpu as pltpu
from jax.experimental.pallas import tpu_sc as plsc

_NUM_SEGMENTS = 1024
_N = 320000
_D = 128

_NC = 2
_NS = 16
_L = 16
_NW = _NC * _NS
_CH = _N // _NW
_CHV = _CH // _L
_HIST = _L * _NUM_SEGMENTS

_GW = 128
_GROWS = 80
_GFIRE = 8


def _sc_partials(out_flat, batch_i32):
    mesh = plsc.VectorSubcoreMesh(core_axis_name="c", subcore_axis_name="s")

    @functools.partial(
        pl.kernel,
        out_type=(
            jax.ShapeDtypeStruct((_NW * _NUM_SEGMENTS // 128, 128), jnp.float32),
            jax.ShapeDtypeStruct((_NW * _NUM_SEGMENTS // 128, 128), jnp.float32),
        ),
        mesh=mesh,
        compiler_params=pltpu.CompilerParams(
            use_tc_tiling_on_sc=False, needs_layout_passes=False
        ),
        scratch_types=[
            pltpu.VMEM((_CH,), jnp.int32),
            pltpu.VMEM((_GROWS, _GW), jnp.int32),
            pltpu.VMEM((_GROWS, _GW), jnp.float32),
            pltpu.VMEM((_HIST // 128, 128), jnp.float32),
            pltpu.VMEM((_HIST // 128, 128), jnp.float32),
            pltpu.VMEM((_NUM_SEGMENTS // 128, 128), jnp.float32),
            pltpu.VMEM((_NUM_SEGMENTS // 128, 128), jnp.float32),
            pltpu.SemaphoreType.DMA,
            pltpu.SemaphoreType.DMA,
            pltpu.SemaphoreType.DMA,
        ],
    )
    def k(flat_hbm, ids_hbm, parts_s, parts_c,
          ids_v, idx_v, vals_g, acc_s, acc_c, red_s, red_c,
          sem_i, sem_a, sem_b):
        wid = lax.axis_index("s") * _NC + lax.axis_index("c")
        base = wid * _CH

        ids_cp = pltpu.async_copy(ids_hbm.at[pl.ds(base, _CH)], ids_v, sem_i)

        lane_iota = lax.iota(jnp.int32, _L)
        last = base + (_CH - 1)
        sems = (sem_a, sem_b)
        ngroups = _GROWS // _GFIRE

        nrows = -(-_CH // _GW)

        def fill_group(g):
            for r in range(_GFIRE):
                j = g * _GFIRE + r
                if j >= nrows:
                    continue
                for kk in range(_GW // _L):
                    elem = base + j * _GW + kk * _L + lane_iota
                    elem = jnp.minimum(elem, last)
                    idx_v[j, pl.ds(kk * _L, _L)] = elem * _D

        def fire_group(g):
            s = sems[g % 2]
            return [
                pltpu.async_copy(
                    flat_hbm.at[idx_v.at[g * _GFIRE + r]],
                    vals_g.at[g * _GFIRE + r], s)
                for r in range(_GFIRE)
                if g * _GFIRE + r < nrows
            ]

        fill_group(0)
        inflight = {0: fire_group(0)}

        zeros = jnp.zeros((_L,), jnp.float32)

        def zero_body(i, _):
            acc_s[i // 8, pl.ds((i % 8) * _L, _L)] = zeros
            acc_c[i // 8, pl.ds((i % 8) * _L, _L)] = zeros
            return None

        lax.fori_loop(0, _HIST // _L, zero_body, None, unroll=8)

        ids_cp.wait()

        lane_off = lane_iota * _NUM_SEGMENTS
        ones = jnp.ones((_L,), jnp.float32)

        def acc_body(t, _):
            ids = ids_v[pl.ds(t * _L, _L)]
            vals = vals_g[t // 8, pl.ds((t % 8) * _L, _L)]
            addr = ids + lane_off
            row = lax.shift_right_logical(addr, 7)
            col = lax.bitwise_and(addr, 127)
            plsc.addupdate_scatter(acc_s, [row, col], vals)
            plsc.addupdate_scatter(acc_c, [row, col], ones)
            return None

        vpg = _GFIRE * _GW // _L
        for g in range(ngroups):
            if g + 1 < ngroups:
                fill_group(g + 1)
                inflight[g + 1] = fire_group(g + 1)
            for cp in inflight.pop(g):
                cp.wait()
            lax.fori_loop(g * vpg, min((g + 1) * vpg, _CHV),
                          acc_body, None, unroll=4)

        nblk = _NUM_SEGMENTS // 128

        def red_body(i, _):
            kk = i // 8
            jj = i % 8
            s = acc_s[kk, pl.ds(jj * _L, _L)]
            c = acc_c[kk, pl.ds(jj * _L, _L)]
            for l in range(1, _L):
                s = s + acc_s[l * nblk + kk, pl.ds(jj * _L, _L)]
                c = c + acc_c[l * nblk + kk, pl.ds(jj * _L, _L)]
            red_s[kk, pl.ds(jj * _L, _L)] = s
            red_c[kk, pl.ds(jj * _L, _L)] = c
            return None

        lax.fori_loop(0, nblk * 8, red_body, None)

        prow = wid * nblk
        pltpu.sync_copy(red_s, parts_s.at[pl.ds(prow, nblk), :])
        pltpu.sync_copy(red_c, parts_c.at[pl.ds(prow, nblk), :])

    return k(out_flat, batch_i32)


def _finish_body(ps_ref, pc_ref, o_ref):
    s = jnp.sum(ps_ref[...].reshape(_NW, 8, 128), axis=0)
    c = jnp.sum(pc_ref[...].reshape(_NW, 8, 128), axis=0)
    o_ref[...] = s / jnp.maximum(c, 1.0)


def kernel(outputs, batch, is_global):
    del is_global
    batch_i32 = batch.astype(jnp.int32)
    parts_s, parts_c = _sc_partials(outputs.reshape(_N * _D), batch_i32)
    score2d = pl.pallas_call(
        _finish_body,
        out_shape=jax.ShapeDtypeStruct((8, 128), jnp.float32),
    )(parts_s, parts_c)
    return score2d.reshape(_NUM_SEGMENTS)
